# Initial kernel scaffold; baseline (speedup 1.0000x reference)
import jax
import jax.numpy as jnp
from jax.experimental import pallas as pl


def kernel(edges_protein, edges_gene, protein_table, gene_table):
    raise NotImplementedError("write your pallas kernel here")


# speedup vs baseline: 1.5003x; 1.5003x over previous
import functools

import jax
import jax.numpy as jnp
from jax import lax
from jax.experimental import pallas as pl
from jax.experimental.pallas import tpu as pltpu
from jax.experimental.pallas import tpu_sc as plsc

_B = 16384
_D = 128
_NC = 2
_NS = 16
_NW = _NC * _NS
_BPW = _B // _NW
_CHUNK = 128
_NCH = _BPW // _CHUNK

_mesh = plsc.VectorSubcoreMesh(core_axis_name="c", subcore_axis_name="s")


@functools.partial(
    pl.kernel,
    mesh=_mesh,
    out_type=(
        jax.ShapeDtypeStruct((_B, _D), jnp.float32),
        jax.ShapeDtypeStruct((_B, _D), jnp.float32),
    ),
    scratch_types=[
        pltpu.VMEM((_NCH, _CHUNK), jnp.int32),
        pltpu.VMEM((_NCH, _CHUNK), jnp.int32),
        pltpu.VMEM((_NCH, _CHUNK, _D), jnp.float32),
        pltpu.SemaphoreType.DMA,
    ],
)
def _bipartite_gather(ep, eg, pt, gt, outp, outg, idxp, idxg, rows, gsem):
    wid = lax.axis_index("s") * _NC + lax.axis_index("c")
    base = wid * _BPW
    pltpu.sync_copy(ep.at[wid], idxp)
    pltpu.sync_copy(eg.at[wid], idxg)

    gathers = [
        pltpu.async_copy(pt.at[idxp.at[j]], rows.at[j], gsem)
        for j in range(_NCH)
    ]
    for j in range(_NCH):
        gathers[j].wait()
        pltpu.sync_copy(rows.at[j], outp.at[pl.ds(base + j * _CHUNK, _CHUNK)])

    gathers = [
        pltpu.async_copy(gt.at[idxg.at[j]], rows.at[j], gsem)
        for j in range(_NCH)
    ]
    for j in range(_NCH):
        gathers[j].wait()
        pltpu.sync_copy(rows.at[j], outg.at[pl.ds(base + j * _CHUNK, _CHUNK)])


def kernel(edges_protein, edges_gene, protein_table, gene_table):
    ep = edges_protein.astype(jnp.int32).reshape(_NW, _NCH, _CHUNK)
    eg = edges_gene.astype(jnp.int32).reshape(_NW, _NCH, _CHUNK)
    return _bipartite_gather(ep, eg, protein_table, gene_table)


# speedup vs baseline: 1.5563x; 1.0374x over previous
import functools

import jax
import jax.numpy as jnp
from jax import lax
from jax.experimental import pallas as pl
from jax.experimental.pallas import tpu as pltpu
from jax.experimental.pallas import tpu_sc as plsc

_B = 16384
_D = 128
_NC = 2
_NS = 16
_NW = _NC * _NS
_BPW = _B // _NW
_CHUNK = 128
_NCH = _BPW // _CHUNK

_NBUF = 7
_NT = 2 * _NCH

_mesh = plsc.VectorSubcoreMesh(core_axis_name="c", subcore_axis_name="s")


@functools.partial(
    pl.kernel,
    mesh=_mesh,
    out_type=(
        jax.ShapeDtypeStruct((_B, _D), jnp.float32),
        jax.ShapeDtypeStruct((_B, _D), jnp.float32),
    ),
    scratch_types=[
        pltpu.VMEM((_NCH, _CHUNK), jnp.int32),
        pltpu.VMEM((_NCH, _CHUNK), jnp.int32),
        pltpu.VMEM((_NBUF, _CHUNK, _D), jnp.float32),
        pltpu.SemaphoreType.DMA,
        pltpu.SemaphoreType.DMA,
    ],
)
def _bipartite_gather(ep, eg, pt, gt, outp, outg, idxp, idxg, rows, gsem, wsem):
    wid = lax.axis_index("s") * _NC + lax.axis_index("c")
    base = wid * _BPW
    pltpu.sync_copy(ep.at[wid], idxp)

    def gather(t, b):
        tbl, idx = (pt, idxp) if t < _NCH else (gt, idxg)
        return pltpu.async_copy(tbl.at[idx.at[t % _NCH]], rows.at[b], gsem)

    def write(t, b):
        out = outp if t < _NCH else outg
        dst = out.at[pl.ds(base + (t % _NCH) * _CHUNK, _CHUNK)]
        return pltpu.async_copy(rows.at[b], dst, wsem)

    gathers = [gather(t, t) for t in range(_NCH)]
    pltpu.sync_copy(eg.at[wid], idxg)
    gathers += [gather(t, t) for t in range(_NCH, _NBUF)]
    writes = []
    for t in range(_NT):
        gathers[t].wait()
        writes.append(write(t, t % _NBUF))
        if t == 0:
            writes[0].wait()
            gathers.append(gather(_NT - 1, 0))
    for t in range(1, _NT):
        writes[t].wait()


def kernel(edges_protein, edges_gene, protein_table, gene_table):
    ep = edges_protein.astype(jnp.int32).reshape(_NW, _NCH, _CHUNK)
    eg = edges_gene.astype(jnp.int32).reshape(_NW, _NCH, _CHUNK)
    return _bipartite_gather(ep, eg, protein_table, gene_table)
